# baseline (device time: 105206 ns/iter reference)
import jax
import jax.numpy as jnp
from jax import lax
from jax.experimental import pallas as pl
from jax.experimental.pallas import tpu as pltpu

N_DEV = 8
B, SQ, SKV, HQ, DH = 2, 256, 2048, 4, 64
D_MODEL = 512
D_HEADS = HQ * DH
SKV_LOC = SKV // N_DEV


def _body(x_ref, wq_ref, k_ref, v_ref, wo_ref, out_ref,
          k_all, v_all, ctx_ref,
          k_send, k_recv, v_send, v_recv):
    my = lax.axis_index("i")
    left = lax.rem(my - 1 + N_DEV, N_DEV)
    right = lax.rem(my + 1, N_DEV)

    barrier = pltpu.get_barrier_semaphore()
    for nbr in (left, right):
        pl.semaphore_signal(barrier, inc=1, device_id=(nbr,),
                            device_id_type=pl.DeviceIdType.MESH)
    pl.semaphore_wait(barrier, 2)

    k_all[my] = k_ref[...]
    v_all[my] = v_ref[...]

    for h in range(N_DEV - 1):
        snd = lax.rem(my - h + N_DEV, N_DEV)
        rcv = lax.rem(my - h - 1 + N_DEV, N_DEV)
        k_rdma = pltpu.make_async_remote_copy(
            src_ref=k_all.at[snd], dst_ref=k_all.at[snd],
            send_sem=k_send.at[snd], recv_sem=k_recv.at[snd],
            device_id=(right,), device_id_type=pl.DeviceIdType.MESH)
        v_rdma = pltpu.make_async_remote_copy(
            src_ref=v_all.at[snd], dst_ref=v_all.at[snd],
            send_sem=v_send.at[snd], recv_sem=v_recv.at[snd],
            device_id=(right,), device_id_type=pl.DeviceIdType.MESH)
        k_rdma.start()
        v_rdma.start()
        k_rdma.wait_send()
        v_rdma.wait_send()
        k_wait = pltpu.make_async_remote_copy(
            src_ref=k_all.at[rcv], dst_ref=k_all.at[rcv],
            send_sem=k_send.at[rcv], recv_sem=k_recv.at[rcv],
            device_id=(right,), device_id_type=pl.DeviceIdType.MESH)
        v_wait = pltpu.make_async_remote_copy(
            src_ref=v_all.at[rcv], dst_ref=v_all.at[rcv],
            send_sem=v_send.at[rcv], recv_sem=v_recv.at[rcv],
            device_id=(right,), device_id_type=pl.DeviceIdType.MESH)
        k_wait.wait_recv()
        v_wait.wait_recv()

    x2 = x_ref[...].reshape(B * SQ, D_MODEL)
    q2 = jnp.dot(x2, wq_ref[...], preferred_element_type=jnp.float32)

    qi = lax.broadcasted_iota(jnp.int32, (SQ, SKV), 0)
    ki = lax.broadcasted_iota(jnp.int32, (SQ, SKV), 1)
    mask = (jnp.abs(qi - ki) <= 128) | (ki < 32) | (qi < 32)

    for b in range(B):
        for hh in range(HQ):
            q = q2[b * SQ:(b + 1) * SQ, hh * DH:(hh + 1) * DH]
            kfull = jnp.concatenate(
                [k_all[kk, b, :, hh * DH:(hh + 1) * DH] for kk in range(N_DEV)],
                axis=0)
            vfull = jnp.concatenate(
                [v_all[kk, b, :, hh * DH:(hh + 1) * DH] for kk in range(N_DEV)],
                axis=0)
            sc = lax.dot_general(q, kfull, (((1,), (1,)), ((), ())),
                                 preferred_element_type=jnp.float32) * 0.125
            sc = jnp.where(mask, sc, -1e9)
            m = jnp.max(sc, axis=1, keepdims=True)
            w = jnp.exp(sc - m)
            den = jnp.sum(w, axis=1, keepdims=True)
            ctx = jnp.dot(w, vfull, preferred_element_type=jnp.float32) / den
            ctx_ref[b, :, hh * DH:(hh + 1) * DH] = ctx

    c2 = ctx_ref[...].reshape(B * SQ, D_HEADS)
    out_ref[...] = jnp.dot(
        c2, wo_ref[...], preferred_element_type=jnp.float32
    ).reshape(B, SQ, D_MODEL)


def kernel(x, Wq, K_ext, V_ext, Wo):
    K2 = K_ext.reshape(B, SKV_LOC, D_HEADS)
    V2 = V_ext.reshape(B, SKV_LOC, D_HEADS)
    return pl.pallas_call(
        _body,
        out_shape=jax.ShapeDtypeStruct((B, SQ, D_MODEL), jnp.float32),
        in_specs=[pl.BlockSpec(memory_space=pltpu.VMEM)] * 5,
        out_specs=pl.BlockSpec(memory_space=pltpu.VMEM),
        scratch_shapes=[
            pltpu.VMEM((N_DEV, B, SKV_LOC, D_HEADS), jnp.float32),
            pltpu.VMEM((N_DEV, B, SKV_LOC, D_HEADS), jnp.float32),
            pltpu.VMEM((B, SQ, D_HEADS), jnp.float32),
            pltpu.SemaphoreType.DMA((N_DEV,)),
            pltpu.SemaphoreType.DMA((N_DEV,)),
            pltpu.SemaphoreType.DMA((N_DEV,)),
            pltpu.SemaphoreType.DMA((N_DEV,)),
        ],
        compiler_params=pltpu.CompilerParams(collective_id=0),
    )(x, Wq, K2, V2, Wo)


# device time: 33027 ns/iter; 3.1855x vs baseline; 3.1855x over previous
import jax
import jax.numpy as jnp
from jax import lax
from jax.experimental import pallas as pl
from jax.experimental.pallas import tpu as pltpu

N_DEV = 8
N_STEPS = 3
B, SQ, SKV, HQ, DH = 2, 256, 2048, 4, 64
D_MODEL = 512
D_HEADS = HQ * DH
SKV_LOC = SKV // N_DEV
P_ROWS = B * SQ + B * HQ


def _body(x_ref, wq_ref, k_ref, v_ref, wo_ref, out_ref,
          work, rbuf, send_sems, recv_sems):
    my = lax.axis_index("i")

    barrier = pltpu.get_barrier_semaphore()
    for s in range(N_STEPS):
        partner = jnp.bitwise_xor(my, 1 << s)
        pl.semaphore_signal(barrier, inc=1, device_id=(partner,),
                            device_id_type=pl.DeviceIdType.MESH)
    pl.semaphore_wait(barrier, N_STEPS)

    x2 = x_ref[...].reshape(B * SQ, D_MODEL)
    q2 = jnp.dot(x2, wq_ref[...], preferred_element_type=jnp.float32)

    ji = lax.broadcasted_iota(jnp.int32, (SKV_LOC, SQ), 0) + my * SKV_LOC
    qi = lax.broadcasted_iota(jnp.int32, (SKV_LOC, SQ), 1)
    mask_t = (jnp.abs(qi - ji) <= 128) | (ji < 32) | (qi < 32)

    for b in range(B):
        for hh in range(HQ):
            q = q2[b * SQ:(b + 1) * SQ, hh * DH:(hh + 1) * DH]
            kc = k_ref[b, :, hh * DH:(hh + 1) * DH]
            vc = v_ref[b, :, hh * DH:(hh + 1) * DH]
            st = lax.dot_general(kc, q, (((1,), (1,)), ((), ())),
                                 preferred_element_type=jnp.float32) * 0.125
            wt = jnp.where(mask_t, jnp.exp(st), 0.0)
            acc = lax.dot_general(wt, vc, (((0,), (0,)), ((), ())),
                                  preferred_element_type=jnp.float32)
            l_row = jnp.sum(wt, axis=0, keepdims=True)
            work[b * SQ:(b + 1) * SQ, hh * DH:(hh + 1) * DH] = acc
            lr = B * SQ + b * HQ + hh
            work[lr:lr + 1, :] = l_row

    for s in range(N_STEPS):
        partner = jnp.bitwise_xor(my, 1 << s)
        rdma = pltpu.make_async_remote_copy(
            src_ref=work, dst_ref=rbuf.at[s],
            send_sem=send_sems.at[s], recv_sem=recv_sems.at[s],
            device_id=(partner,), device_id_type=pl.DeviceIdType.MESH)
        rdma.start()
        rdma.wait_recv()
        rdma.wait_send()
        work[...] = work[...] + rbuf[s]

    eye = (lax.broadcasted_iota(jnp.int32, (SQ, SQ), 0)
           == lax.broadcasted_iota(jnp.int32, (SQ, SQ), 1))
    for b in range(B):
        for hh in range(HQ):
            lr = B * SQ + b * HQ + hh
            l_row = work[lr:lr + 1, :]
            l_col = jnp.sum(jnp.where(eye, jnp.broadcast_to(l_row, (SQ, SQ)), 0.0),
                            axis=1, keepdims=True)
            acc = work[b * SQ:(b + 1) * SQ, hh * DH:(hh + 1) * DH]
            ctx = acc / l_col
            work[b * SQ:(b + 1) * SQ, hh * DH:(hh + 1) * DH] = ctx

    c2 = work[0:B * SQ, :]
    out_ref[...] = jnp.dot(
        c2, wo_ref[...], preferred_element_type=jnp.float32
    ).reshape(B, SQ, D_MODEL)


def kernel(x, Wq, K_ext, V_ext, Wo):
    K2 = K_ext.reshape(B, SKV_LOC, D_HEADS)
    V2 = V_ext.reshape(B, SKV_LOC, D_HEADS)
    return pl.pallas_call(
        _body,
        out_shape=jax.ShapeDtypeStruct((B, SQ, D_MODEL), jnp.float32),
        in_specs=[pl.BlockSpec(memory_space=pltpu.VMEM)] * 5,
        out_specs=pl.BlockSpec(memory_space=pltpu.VMEM),
        scratch_shapes=[
            pltpu.VMEM((P_ROWS, D_HEADS), jnp.float32),
            pltpu.VMEM((N_STEPS, P_ROWS, D_HEADS), jnp.float32),
            pltpu.SemaphoreType.DMA((N_STEPS,)),
            pltpu.SemaphoreType.DMA((N_STEPS,)),
        ],
        compiler_params=pltpu.CompilerParams(collective_id=0),
    )(x, Wq, K2, V2, Wo)


# device time: 21010 ns/iter; 5.0074x vs baseline; 1.5720x over previous
import jax
import jax.numpy as jnp
from jax import lax
from jax.experimental import pallas as pl
from jax.experimental.pallas import tpu as pltpu

N_DEV = 8
N_STEPS = 3
MASKS = (1, 3, 4)
B, SQ, SKV, HQ, DH = 2, 256, 2048, 4, 64
D_MODEL = 512
D_HEADS = HQ * DH
SKV_LOC = SKV // N_DEV
P_ROWS = B * SQ + B * HQ
FRAGS = ((0, 176), (176, 176), (352, 168))


def _body(x_ref, wq_ref, k_ref, v_ref, wo_ref, out_ref,
          work, rbuf, send_sems, recv_sems):
    my = lax.axis_index("i")

    barrier = pltpu.get_barrier_semaphore()
    for m in MASKS:
        pl.semaphore_signal(barrier, inc=1,
                            device_id=(jnp.bitwise_xor(my, m),),
                            device_id_type=pl.DeviceIdType.MESH)
    pl.semaphore_wait(barrier, N_STEPS)

    x2 = x_ref[...].reshape(B * SQ, D_MODEL)
    q2 = jnp.dot(x2, wq_ref[...], preferred_element_type=jnp.float32)

    ji = lax.broadcasted_iota(jnp.int32, (SKV_LOC, SQ), 0) + my * SKV_LOC
    qi = lax.broadcasted_iota(jnp.int32, (SKV_LOC, SQ), 1)
    mask_t = (jnp.abs(qi - ji) <= 128) | (ji < 32) | (qi < 32)

    for b in range(B):
        for hh in range(HQ):
            q = q2[b * SQ:(b + 1) * SQ, hh * DH:(hh + 1) * DH]
            kc = k_ref[b, :, hh * DH:(hh + 1) * DH]
            vc = v_ref[b, :, hh * DH:(hh + 1) * DH]
            st = lax.dot_general(kc, q, (((1,), (1,)), ((), ())),
                                 preferred_element_type=jnp.float32) * 0.125
            wt = jnp.where(mask_t, jnp.exp(st), 0.0)
            acc = lax.dot_general(wt, vc, (((0,), (0,)), ((), ())),
                                  preferred_element_type=jnp.float32)
            l_row = jnp.sum(wt, axis=0, keepdims=True)
            work[b * SQ:(b + 1) * SQ, hh * DH:(hh + 1) * DH] = acc
            lr = B * SQ + b * HQ + hh
            work[lr:lr + 1, :] = l_row

    for s in range(N_STEPS):
        rdmas = []
        for f, (r0, rl) in enumerate(FRAGS):
            partner = jnp.bitwise_xor(my, MASKS[(f + s) % N_STEPS])
            rdma = pltpu.make_async_remote_copy(
                src_ref=work.at[r0:r0 + rl],
                dst_ref=rbuf.at[s, r0:r0 + rl],
                send_sem=send_sems.at[s, f], recv_sem=recv_sems.at[s, f],
                device_id=(partner,), device_id_type=pl.DeviceIdType.MESH)
            rdma.start()
            rdmas.append(rdma)
        for rdma in rdmas:
            rdma.wait_recv()
            rdma.wait_send()
        work[...] = work[...] + rbuf[s]

    eye = (lax.broadcasted_iota(jnp.int32, (SQ, SQ), 0)
           == lax.broadcasted_iota(jnp.int32, (SQ, SQ), 1))
    for b in range(B):
        for hh in range(HQ):
            lr = B * SQ + b * HQ + hh
            l_row = work[lr:lr + 1, :]
            l_col = jnp.sum(jnp.where(eye, jnp.broadcast_to(l_row, (SQ, SQ)), 0.0),
                            axis=1, keepdims=True)
            acc = work[b * SQ:(b + 1) * SQ, hh * DH:(hh + 1) * DH]
            ctx = acc / l_col
            work[b * SQ:(b + 1) * SQ, hh * DH:(hh + 1) * DH] = ctx

    c2 = work[0:B * SQ, :]
    out_ref[...] = jnp.dot(
        c2, wo_ref[...], preferred_element_type=jnp.float32
    ).reshape(B, SQ, D_MODEL)


def kernel(x, Wq, K_ext, V_ext, Wo):
    K2 = K_ext.reshape(B, SKV_LOC, D_HEADS)
    V2 = V_ext.reshape(B, SKV_LOC, D_HEADS)
    return pl.pallas_call(
        _body,
        out_shape=jax.ShapeDtypeStruct((B, SQ, D_MODEL), jnp.float32),
        in_specs=[pl.BlockSpec(memory_space=pltpu.VMEM)] * 5,
        out_specs=pl.BlockSpec(memory_space=pltpu.VMEM),
        scratch_shapes=[
            pltpu.VMEM((P_ROWS, D_HEADS), jnp.float32),
            pltpu.VMEM((N_STEPS, P_ROWS, D_HEADS), jnp.float32),
            pltpu.SemaphoreType.DMA((N_STEPS, N_STEPS)),
            pltpu.SemaphoreType.DMA((N_STEPS, N_STEPS)),
        ],
        compiler_params=pltpu.CompilerParams(collective_id=0),
    )(x, Wq, K2, V2, Wo)


# device time: 18141 ns/iter; 5.7993x vs baseline; 1.1582x over previous
import jax
import jax.numpy as jnp
from jax import lax
from jax.experimental import pallas as pl
from jax.experimental.pallas import tpu as pltpu

N_DEV = 8
N_STEPS = 3
MASKS = (1, 3, 4)
B, SQ, SKV, HQ, DH = 2, 256, 2048, 4, 64
D_MODEL = 512
D_HEADS = HQ * DH
SKV_LOC = SKV // N_DEV
P_ROWS = B * SQ + B * HQ
FRAGS = ((0, 176), (176, 176), (352, 168))


def _body(x_ref, wq_ref, k_ref, v_ref, wo_ref, out_ref,
          work, sbuf, rbuf, send_sems, recv_sems):
    my = lax.axis_index("i")

    barrier = pltpu.get_barrier_semaphore()
    for m in MASKS:
        pl.semaphore_signal(barrier, inc=1,
                            device_id=(jnp.bitwise_xor(my, m),),
                            device_id_type=pl.DeviceIdType.MESH)
    pl.semaphore_wait(barrier, N_STEPS)

    x2 = x_ref[...].reshape(B * SQ, D_MODEL).astype(jnp.bfloat16)
    q2 = jnp.dot(x2, wq_ref[...].astype(jnp.bfloat16),
                 preferred_element_type=jnp.float32)

    ji = lax.broadcasted_iota(jnp.int32, (SKV_LOC, SQ), 0) + my * SKV_LOC
    qi = lax.broadcasted_iota(jnp.int32, (SKV_LOC, SQ), 1)
    mask_t = (jnp.abs(qi - ji) <= 128) | (ji < 32) | (qi < 32)

    for b in range(B):
        for hh in range(HQ):
            q = q2[b * SQ:(b + 1) * SQ, hh * DH:(hh + 1) * DH]
            kc = k_ref[b, :, hh * DH:(hh + 1) * DH]
            vc = v_ref[b, :, hh * DH:(hh + 1) * DH]
            st = lax.dot_general(kc, q, (((1,), (1,)), ((), ())),
                                 preferred_element_type=jnp.float32) * 0.125
            wt = jnp.where(mask_t, jnp.exp(st), 0.0)
            acc = lax.dot_general(wt, vc, (((0,), (0,)), ((), ())),
                                  preferred_element_type=jnp.float32)
            l_row = jnp.sum(wt, axis=0, keepdims=True)
            work[b * SQ:(b + 1) * SQ, hh * DH:(hh + 1) * DH] = acc
            lr = B * SQ + b * HQ + hh
            work[lr:lr + 1, :] = l_row

    for s in range(N_STEPS):
        sbuf[...] = work[...].astype(jnp.bfloat16)
        rdmas = []
        for f, (r0, rl) in enumerate(FRAGS):
            partner = jnp.bitwise_xor(my, MASKS[(f + s) % N_STEPS])
            rdma = pltpu.make_async_remote_copy(
                src_ref=sbuf.at[r0:r0 + rl],
                dst_ref=rbuf.at[s, r0:r0 + rl],
                send_sem=send_sems.at[s, f], recv_sem=recv_sems.at[s, f],
                device_id=(partner,), device_id_type=pl.DeviceIdType.MESH)
            rdma.start()
            rdmas.append(rdma)
        for rdma in rdmas:
            rdma.wait_recv()
            rdma.wait_send()
        work[...] = work[...] + rbuf[s].astype(jnp.float32)

    eye = (lax.broadcasted_iota(jnp.int32, (SQ, SQ), 0)
           == lax.broadcasted_iota(jnp.int32, (SQ, SQ), 1))
    for b in range(B):
        for hh in range(HQ):
            lr = B * SQ + b * HQ + hh
            l_row = work[lr:lr + 1, :]
            l_col = jnp.sum(jnp.where(eye, jnp.broadcast_to(l_row, (SQ, SQ)), 0.0),
                            axis=1, keepdims=True)
            acc = work[b * SQ:(b + 1) * SQ, hh * DH:(hh + 1) * DH]
            ctx = acc / l_col
            work[b * SQ:(b + 1) * SQ, hh * DH:(hh + 1) * DH] = ctx

    c2 = work[0:B * SQ, :].astype(jnp.bfloat16)
    out_ref[...] = jnp.dot(
        c2, wo_ref[...].astype(jnp.bfloat16),
        preferred_element_type=jnp.float32,
    ).reshape(B, SQ, D_MODEL)


def kernel(x, Wq, K_ext, V_ext, Wo):
    K2 = K_ext.reshape(B, SKV_LOC, D_HEADS)
    V2 = V_ext.reshape(B, SKV_LOC, D_HEADS)
    return pl.pallas_call(
        _body,
        out_shape=jax.ShapeDtypeStruct((B, SQ, D_MODEL), jnp.float32),
        in_specs=[pl.BlockSpec(memory_space=pltpu.VMEM)] * 5,
        out_specs=pl.BlockSpec(memory_space=pltpu.VMEM),
        scratch_shapes=[
            pltpu.VMEM((P_ROWS, D_HEADS), jnp.float32),
            pltpu.VMEM((P_ROWS, D_HEADS), jnp.bfloat16),
            pltpu.VMEM((N_STEPS, P_ROWS, D_HEADS), jnp.bfloat16),
            pltpu.SemaphoreType.DMA((N_STEPS, N_STEPS)),
            pltpu.SemaphoreType.DMA((N_STEPS, N_STEPS)),
        ],
        compiler_params=pltpu.CompilerParams(collective_id=0),
    )(x, Wq, K2, V2, Wo)


# device time: 18113 ns/iter; 5.8083x vs baseline; 1.0015x over previous
import jax
import jax.numpy as jnp
from jax import lax
from jax.experimental import pallas as pl
from jax.experimental.pallas import tpu as pltpu

N_DEV = 8
N_STEPS = 3
MASKS = (1, 3, 4)
B, SQ, SKV, HQ, DH = 2, 256, 2048, 4, 64
D_MODEL = 512
D_HEADS = HQ * DH
SKV_LOC = SKV // N_DEV
P_ROWS = B * SQ + B * HQ
FRAGS = ((0, 176), (176, 176), (352, 168))


def _body(x_ref, wq_ref, k_ref, v_ref, wo_ref, out_ref,
          work, sbuf, rbuf, send_sems, recv_sems):
    my = lax.axis_index("i")

    x2 = x_ref[...].reshape(B * SQ, D_MODEL).astype(jnp.bfloat16)
    q2 = jnp.dot(x2, wq_ref[...].astype(jnp.bfloat16),
                 preferred_element_type=jnp.float32)

    ji = lax.broadcasted_iota(jnp.int32, (SKV_LOC, SQ), 0) + my * SKV_LOC
    qi = lax.broadcasted_iota(jnp.int32, (SKV_LOC, SQ), 1)
    mask_t = (jnp.abs(qi - ji) <= 128) | (ji < 32) | (qi < 32)

    for b in range(B):
        for hh in range(HQ):
            q = q2[b * SQ:(b + 1) * SQ, hh * DH:(hh + 1) * DH]
            kc = k_ref[b, :, hh * DH:(hh + 1) * DH].astype(jnp.bfloat16)
            vc = v_ref[b, :, hh * DH:(hh + 1) * DH].astype(jnp.bfloat16)
            st = lax.dot_general(kc, q.astype(jnp.bfloat16),
                                 (((1,), (1,)), ((), ())),
                                 preferred_element_type=jnp.float32) * 0.125
            wt = jnp.where(mask_t, jnp.exp(st), 0.0)
            acc = lax.dot_general(wt.astype(jnp.bfloat16), vc,
                                  (((0,), (0,)), ((), ())),
                                  preferred_element_type=jnp.float32)
            l_row = jnp.sum(wt, axis=0, keepdims=True)
            work[b * SQ:(b + 1) * SQ, hh * DH:(hh + 1) * DH] = acc
            lr = B * SQ + b * HQ + hh
            work[lr:lr + 1, :] = l_row

    barrier = pltpu.get_barrier_semaphore()
    for m in MASKS:
        pl.semaphore_signal(barrier, inc=1,
                            device_id=(jnp.bitwise_xor(my, m),),
                            device_id_type=pl.DeviceIdType.MESH)
    pl.semaphore_wait(barrier, N_STEPS)

    for s in range(N_STEPS):
        sbuf[...] = work[...].astype(jnp.bfloat16)
        rdmas = []
        for f, (r0, rl) in enumerate(FRAGS):
            partner = jnp.bitwise_xor(my, MASKS[(f + s) % N_STEPS])
            rdma = pltpu.make_async_remote_copy(
                src_ref=sbuf.at[r0:r0 + rl],
                dst_ref=rbuf.at[s, r0:r0 + rl],
                send_sem=send_sems.at[s, f], recv_sem=recv_sems.at[s, f],
                device_id=(partner,), device_id_type=pl.DeviceIdType.MESH)
            rdma.start()
            rdmas.append(rdma)
        for rdma in rdmas:
            rdma.wait_recv()
            rdma.wait_send()
        work[...] = work[...] + rbuf[s].astype(jnp.float32)

    eye = (lax.broadcasted_iota(jnp.int32, (SQ, SQ), 0)
           == lax.broadcasted_iota(jnp.int32, (SQ, SQ), 1)).astype(jnp.float32)
    l_rows = work[B * SQ:P_ROWS, :]
    l_cols = lax.dot_general(eye, l_rows, (((1,), (1,)), ((), ())),
                             preferred_element_type=jnp.float32)
    inv = 1.0 / l_cols
    for b in range(B):
        scale = jnp.concatenate(
            [jnp.broadcast_to(inv[:, b * HQ + hh:b * HQ + hh + 1], (SQ, DH))
             for hh in range(HQ)], axis=1)
        work[b * SQ:(b + 1) * SQ, :] = work[b * SQ:(b + 1) * SQ, :] * scale

    c2 = work[0:B * SQ, :].astype(jnp.bfloat16)
    out_ref[...] = jnp.dot(
        c2, wo_ref[...].astype(jnp.bfloat16),
        preferred_element_type=jnp.float32,
    ).reshape(B, SQ, D_MODEL)


def kernel(x, Wq, K_ext, V_ext, Wo):
    K2 = K_ext.reshape(B, SKV_LOC, D_HEADS)
    V2 = V_ext.reshape(B, SKV_LOC, D_HEADS)
    return pl.pallas_call(
        _body,
        out_shape=jax.ShapeDtypeStruct((B, SQ, D_MODEL), jnp.float32),
        in_specs=[pl.BlockSpec(memory_space=pltpu.VMEM)] * 5,
        out_specs=pl.BlockSpec(memory_space=pltpu.VMEM),
        scratch_shapes=[
            pltpu.VMEM((P_ROWS, D_HEADS), jnp.float32),
            pltpu.VMEM((P_ROWS, D_HEADS), jnp.bfloat16),
            pltpu.VMEM((N_STEPS, P_ROWS, D_HEADS), jnp.bfloat16),
            pltpu.SemaphoreType.DMA((N_STEPS, N_STEPS)),
            pltpu.SemaphoreType.DMA((N_STEPS, N_STEPS)),
        ],
        compiler_params=pltpu.CompilerParams(collective_id=0),
    )(x, Wq, K2, V2, Wo)


# device time: 16443 ns/iter; 6.3982x vs baseline; 1.1016x over previous
import jax
import jax.numpy as jnp
from jax import lax
from jax.experimental import pallas as pl
from jax.experimental.pallas import tpu as pltpu

N_DEV = 8
N_STEPS = 3
MASKS = (1, 3, 4)
B, SQ, SKV, HQ, DH = 2, 256, 2048, 4, 64
D_MODEL = 512
D_HEADS = HQ * DH
SKV_LOC = SKV // N_DEV
FR = 88


def _body(x_ref, wq_ref, k_ref, v_ref, wo_ref, out_ref,
          w00, w01, w02, w10, w11, w12,
          r00, r01, r02, r10, r11, r12,
          send_sems, recv_sems):
    my = lax.axis_index("i")
    works = ((w00, w01, w02), (w10, w11, w12))
    rbufs = ((r00, r01, r02), (r10, r11, r12))

    barrier = pltpu.get_barrier_semaphore()
    for m in MASKS:
        pl.semaphore_signal(barrier, inc=1,
                            device_id=(jnp.bitwise_xor(my, m),),
                            device_id_type=pl.DeviceIdType.MESH)

    x2 = x_ref[...].reshape(B * SQ, D_MODEL).astype(jnp.bfloat16)
    q2 = jnp.dot(x2, wq_ref[...].astype(jnp.bfloat16),
                 preferred_element_type=jnp.float32).astype(jnp.bfloat16)
    wo_bf = wo_ref[...].astype(jnp.bfloat16)

    ji = lax.broadcasted_iota(jnp.int32, (SKV_LOC, SQ), 0) + my * SKV_LOC
    qi = lax.broadcasted_iota(jnp.int32, (SKV_LOC, SQ), 1)
    mask_t = (jnp.abs(qi - ji) <= 128) | (ji < 32) | (qi < 32)

    def partials(b):
        f0, f1, f2 = works[b]
        for hh in range(HQ):
            cs = slice(hh * DH, (hh + 1) * DH)
            q = q2[b * SQ:(b + 1) * SQ, cs]
            kc = k_ref[b, :, cs].astype(jnp.bfloat16)
            vc = v_ref[b, :, cs].astype(jnp.bfloat16)
            st = lax.dot_general(kc, q, (((1,), (1,)), ((), ())),
                                 preferred_element_type=jnp.float32) * 0.125
            wt = jnp.where(mask_t, jnp.exp(st), 0.0)
            acc = lax.dot_general(wt.astype(jnp.bfloat16), vc,
                                  (((0,), (0,)), ((), ())),
                                  preferred_element_type=jnp.float32)
            accb = acc.astype(jnp.bfloat16)
            l_row = jnp.sum(wt, axis=0, keepdims=True)
            f0[:, cs] = accb[0:FR, :]
            f1[:, cs] = accb[FR:2 * FR, :]
            f2[0:80, cs] = accb[2 * FR:SQ, :]
            f2[80 + hh:81 + hh, :] = l_row.astype(jnp.bfloat16)
        f2[84:FR, :] = jnp.zeros((4, D_HEADS), jnp.bfloat16)

    def start(b, f, s):
        partner = jnp.bitwise_xor(my, MASKS[(f + s) % N_STEPS])
        rdma = pltpu.make_async_remote_copy(
            src_ref=works[b][f], dst_ref=rbufs[b][f].at[s],
            send_sem=send_sems.at[s % 2, b * 3 + f],
            recv_sem=recv_sems.at[s % 2, b * 3 + f],
            device_id=(partner,), device_id_type=pl.DeviceIdType.MESH)
        rdma.start()
        return rdma

    def finish(rdma, b, f, s):
        rdma.wait_recv()
        rdma.wait_send()
        works[b][f][...] = works[b][f][...] + rbufs[b][f][s]

    eye = (lax.broadcasted_iota(jnp.int32, (SQ, SQ), 0)
           == lax.broadcasted_iota(jnp.int32, (SQ, SQ), 1)).astype(jnp.bfloat16)

    def norm_proj(b):
        f0, f1, f2 = works[b]
        lrows = f2[80:84, :]
        lcols = lax.dot_general(eye, lrows, (((1,), (1,)), ((), ())),
                                preferred_element_type=jnp.float32)
        inv = 1.0 / lcols
        scale = jnp.concatenate(
            [jnp.broadcast_to(inv[:, hh:hh + 1], (SQ, DH)) for hh in range(HQ)],
            axis=1)
        accf = jnp.concatenate([f0[...], f1[...], f2[0:80, :]], axis=0)
        ctx = (accf.astype(jnp.float32) * scale).astype(jnp.bfloat16)
        out_ref[b, :, :] = jnp.dot(ctx, wo_bf,
                                   preferred_element_type=jnp.float32)

    partials(0)
    pl.semaphore_wait(barrier, N_STEPS)
    d0 = [start(0, f, 0) for f in range(3)]
    partials(1)
    d1 = [start(1, f, 0) for f in range(3)]
    for s in range(N_STEPS - 1):
        for f in range(3):
            finish(d0[f], 0, f, s)
            d0[f] = start(0, f, s + 1)
        for f in range(3):
            finish(d1[f], 1, f, s)
            d1[f] = start(1, f, s + 1)
    for f in range(3):
        finish(d0[f], 0, f, N_STEPS - 1)
    norm_proj(0)
    for f in range(3):
        finish(d1[f], 1, f, N_STEPS - 1)
    norm_proj(1)


def kernel(x, Wq, K_ext, V_ext, Wo):
    bf = jnp.bfloat16
    K2 = K_ext.reshape(B, SKV_LOC, D_HEADS)
    V2 = V_ext.reshape(B, SKV_LOC, D_HEADS)
    return pl.pallas_call(
        _body,
        out_shape=jax.ShapeDtypeStruct((B, SQ, D_MODEL), jnp.float32),
        in_specs=[pl.BlockSpec(memory_space=pltpu.VMEM)] * 5,
        out_specs=pl.BlockSpec(memory_space=pltpu.VMEM),
        scratch_shapes=(
            [pltpu.VMEM((FR, D_HEADS), bf)] * 6 +
            [pltpu.VMEM((N_STEPS, FR, D_HEADS), bf)] * 6 +
            [pltpu.SemaphoreType.DMA((2, 6)),
             pltpu.SemaphoreType.DMA((2, 6))]
        ),
        compiler_params=pltpu.CompilerParams(collective_id=0),
    )(x, Wq, K2, V2, Wo)
